# baseline (speedup 1.0000x reference)
import functools

import jax
import jax.numpy as jnp
from jax import lax
from jax.experimental import pallas as pl
from jax.experimental.pallas import tpu as pltpu
from jax.experimental.pallas import tpu_sc as plsc

NUM_CORES = 2
NUM_SUBCORES = 16
NUM_WORKERS = NUM_CORES * NUM_SUBCORES

CHUNK = 512


def _make_gather(total, dim, dtype):
    assert total % (8 * NUM_WORKERS) == 0
    per_w = total // NUM_WORKERS
    assert per_w % CHUNK == 0
    n_chunks = per_w // CHUNK
    mesh = plsc.VectorSubcoreMesh(core_axis_name="c", subcore_axis_name="s")

    @functools.partial(
        pl.kernel,
        mesh=mesh,
        out_type=jax.ShapeDtypeStruct((total, dim), dtype),
        scratch_types=[
            pltpu.VMEM((CHUNK,), jnp.int32),
            pltpu.VMEM((CHUNK, dim), dtype),
            pltpu.SemaphoreType.DMA,
        ],
        compiler_params=pltpu.CompilerParams(use_tc_tiling_on_sc=False),
    )
    def gather_kernel(idx_hbm, table_hbm, out_hbm, idx_v, rows_v, sem):
        wid = lax.axis_index("s") * NUM_CORES + lax.axis_index("c")
        base = wid * per_w

        def body(g, carry):
            off = pl.multiple_of(base + g * CHUNK, CHUNK)
            pltpu.sync_copy(idx_hbm.at[pl.ds(off, CHUNK)], idx_v)
            pltpu.async_copy(table_hbm.at[idx_v], rows_v, sem).wait()
            pltpu.sync_copy(rows_v, out_hbm.at[pl.ds(off, CHUNK)])
            return carry

        lax.fori_loop(0, n_chunks, body, 0)

    return gather_kernel


def kernel(token_ids, embedding_matrix):
    batch, seq = token_ids.shape
    num_rows, dim = embedding_matrix.shape
    flat_ids = token_ids.reshape(batch * seq)
    fn = _make_gather(batch * seq, dim, embedding_matrix.dtype)
    out = fn(flat_ids, embedding_matrix)
    return out.reshape(batch, seq, dim)


# speedup vs baseline: 1.0423x; 1.0423x over previous
import functools

import jax
import jax.numpy as jnp
from jax import lax
from jax.experimental import pallas as pl
from jax.experimental.pallas import tpu as pltpu
from jax.experimental.pallas import tpu_sc as plsc

NUM_CORES = 2
NUM_SUBCORES = 16
NUM_WORKERS = NUM_CORES * NUM_SUBCORES

CHUNK = 512
NBUF = 2


def _make_gather(total, dim, dtype):
    assert total % (NUM_WORKERS * CHUNK * NBUF) == 0
    per_w = total // NUM_WORKERS
    n_chunks = per_w // CHUNK
    n_outer = n_chunks // NBUF
    mesh = plsc.VectorSubcoreMesh(core_axis_name="c", subcore_axis_name="s")

    @functools.partial(
        pl.kernel,
        mesh=mesh,
        out_type=jax.ShapeDtypeStruct((total, dim), dtype),
        scratch_types=[
            pltpu.VMEM((per_w,), jnp.int32),
            [pltpu.VMEM((CHUNK, dim), dtype) for _ in range(NBUF)],
            pltpu.SemaphoreType.DMA,
            [pltpu.SemaphoreType.DMA for _ in range(NBUF)],
        ],
        compiler_params=pltpu.CompilerParams(use_tc_tiling_on_sc=False),
    )
    def gather_kernel(idx_hbm, table_hbm, out_hbm, idx_v, rows, gsem, wsems):
        wid = lax.axis_index("s") * NUM_CORES + lax.axis_index("c")
        base = pl.multiple_of(wid * per_w, CHUNK)
        pltpu.sync_copy(idx_hbm.at[pl.ds(base, per_w)], idx_v)

        def do_chunk(g, b, first):
            off = pl.multiple_of(g * CHUNK, CHUNK)
            if not first:
                pltpu.make_async_copy(
                    rows[b], out_hbm.at[pl.ds(0, CHUNK)], wsems[b]
                ).wait()
            pltpu.async_copy(
                table_hbm.at[idx_v.at[pl.ds(off, CHUNK)]], rows[b], gsem
            ).wait()
            pltpu.async_copy(
                rows[b], out_hbm.at[pl.ds(base + off, CHUNK)], wsems[b]
            )

        for b in range(NBUF):
            do_chunk(b, b, first=True)

        def outer(i, carry):
            for b in range(NBUF):
                do_chunk(i * NBUF + b, b, first=False)
            return carry

        lax.fori_loop(1, n_outer, outer, 0)

        for b in range(NBUF):
            pltpu.make_async_copy(
                rows[b], out_hbm.at[pl.ds(0, CHUNK)], wsems[b]
            ).wait()

    return gather_kernel


def kernel(token_ids, embedding_matrix):
    batch, seq = token_ids.shape
    num_rows, dim = embedding_matrix.shape
    flat_ids = token_ids.reshape(batch * seq)
    fn = _make_gather(batch * seq, dim, embedding_matrix.dtype)
    out = fn(flat_ids, embedding_matrix)
    return out.reshape(batch, seq, dim)


# speedup vs baseline: 1.2708x; 1.2193x over previous
import functools

import jax
import jax.numpy as jnp
from jax import lax
from jax.experimental import pallas as pl
from jax.experimental.pallas import tpu as pltpu
from jax.experimental.pallas import tpu_sc as plsc

NUM_CORES = 2
NUM_SUBCORES = 16
NUM_WORKERS = NUM_CORES * NUM_SUBCORES

CHUNK = 256
NBUF = 2


def _make_gather(total, dim, dtype):
    assert total % (NUM_WORKERS * CHUNK * NBUF) == 0
    per_w = total // NUM_WORKERS
    n_chunks = per_w // CHUNK
    n_outer = n_chunks // NBUF
    mesh = plsc.VectorSubcoreMesh(core_axis_name="c", subcore_axis_name="s")

    @functools.partial(
        pl.kernel,
        mesh=mesh,
        out_type=jax.ShapeDtypeStruct((total, dim), dtype),
        scratch_types=[
            pltpu.VMEM((per_w,), jnp.int32),
            [pltpu.VMEM((CHUNK, dim), dtype) for _ in range(NBUF)],
            pltpu.SemaphoreType.DMA,
            [pltpu.SemaphoreType.DMA for _ in range(NBUF)],
        ],
    )
    def gather_kernel(idx_hbm, table_hbm, out_hbm, idx_v, rows, gsem, wsems):
        wid = lax.axis_index("s") * NUM_CORES + lax.axis_index("c")
        base = pl.multiple_of(wid * per_w, CHUNK)
        pltpu.sync_copy(idx_hbm.at[pl.ds(base, per_w)], idx_v)

        def do_chunk(g, b, first):
            off = pl.multiple_of(g * CHUNK, CHUNK)
            if not first:
                pltpu.make_async_copy(
                    rows[b], out_hbm.at[pl.ds(0, CHUNK)], wsems[b]
                ).wait()
            pltpu.async_copy(
                table_hbm.at[idx_v.at[pl.ds(off, CHUNK)]], rows[b], gsem
            ).wait()
            pltpu.async_copy(
                rows[b], out_hbm.at[pl.ds(base + off, CHUNK)], wsems[b]
            )

        for b in range(NBUF):
            do_chunk(b, b, first=True)

        def outer(i, carry):
            for b in range(NBUF):
                do_chunk(i * NBUF + b, b, first=False)
            return carry

        lax.fori_loop(1, n_outer, outer, 0)

        for b in range(NBUF):
            pltpu.make_async_copy(
                rows[b], out_hbm.at[pl.ds(0, CHUNK)], wsems[b]
            ).wait()

    return gather_kernel


def kernel(token_ids, embedding_matrix):
    batch, seq = token_ids.shape
    num_rows, dim = embedding_matrix.shape
    pad_dim = 128
    flat_ids = token_ids.reshape(batch * seq)
    table128 = jnp.pad(embedding_matrix, ((0, 0), (0, pad_dim - dim)))
    fn = _make_gather(batch * seq, pad_dim, embedding_matrix.dtype)
    out = fn(flat_ids, table128)
    return out[:, :dim].reshape(batch, seq, dim)
